# baseline (speedup 1.0000x reference)
import functools

import jax
import jax.numpy as jnp
from jax import lax
from jax.experimental import pallas as pl
from jax.experimental.pallas import tpu as pltpu
from jax.experimental.pallas import tpu_sc as plsc

_CHUNK_IDX = 128
_LANES = 16


def _gather_kernel(B, V, D, NW, b_per_w, n_idx_chunks):
    mesh = plsc.VectorSubcoreMesh(core_axis_name="c", subcore_axis_name="s")
    n_groups = b_per_w // _LANES
    quarter = V // 4

    @functools.partial(
        pl.kernel,
        mesh=mesh,
        out_type=jax.ShapeDtypeStruct((B, D), jnp.float32),
        scratch_types=[
            pltpu.VMEM((n_idx_chunks, _CHUNK_IDX), jnp.int32),
            pltpu.VMEM((n_idx_chunks, _CHUNK_IDX), jnp.int32),
            pltpu.VMEM((n_idx_chunks, _CHUNK_IDX), jnp.int32),
            pltpu.VMEM((_CHUNK_IDX, 2 * D), jnp.float32),
            pltpu.VMEM((_CHUNK_IDX, 2 * D), jnp.float32),
            pltpu.VMEM((_CHUNK_IDX, D), jnp.float32),
            pltpu.SemaphoreType.DMA,
        ],
    )
    def k(idx_hbm, lina_hbm, linb_hbm, out_hbm, idx_v, pa_v, pb_v, rowsa_v, rowsb_v, out_v, sem):
        nc = plsc.get_sparse_core_info().num_cores
        wid = lax.axis_index("s") * nc + lax.axis_index("c")
        row_base = wid * n_idx_chunks
        pltpu.sync_copy(idx_hbm.at[pl.ds(row_base, n_idx_chunks)], idx_v)

        per_row = _CHUNK_IDX // _LANES

        def shift_group(g, carry):
            r = g // per_row
            col = (g % per_row) * _LANES
            p = lax.shift_right_logical(idx_v[r, pl.ds(col, _LANES)], 1)
            pa_v[r, pl.ds(col, _LANES)] = jnp.where(p >= quarter, 0, p)
            pb_v[r, pl.ds(col, _LANES)] = jnp.where(p >= quarter, p - quarter, 0)
            return carry

        lax.fori_loop(0, n_groups, shift_group, 0)

        groups_per_chunk = _CHUNK_IDX // _LANES
        for hh in range(n_idx_chunks):
            ca = pltpu.async_copy(lina_hbm.at[pa_v.at[hh]], rowsa_v, sem)
            pltpu.async_copy(linb_hbm.at[pb_v.at[hh]], rowsb_v, sem).wait()
            ca.wait()

            def select_group(g, carry, hh=hh):
                col = g * _LANES
                vec = idx_v[hh, pl.ds(col, _LANES)]
                for j in range(_LANES):
                    i = g * _LANES + j
                    r = vec[j]
                    hi = r >= 2 * quarter
                    src = (r & 1) * D
                    for kk in range(D // _LANES):
                        va = rowsa_v[i, pl.ds(src + kk * _LANES, _LANES)]
                        vb = rowsb_v[i, pl.ds(src + kk * _LANES, _LANES)]
                        out_v[i, pl.ds(kk * _LANES, _LANES)] = jnp.where(hi, vb, va)
                return carry

            lax.fori_loop(0, groups_per_chunk, select_group, 0)
            pltpu.sync_copy(
                out_v,
                out_hbm.at[pl.ds(wid * b_per_w + hh * _CHUNK_IDX, _CHUNK_IDX)],
            )

    return k


def kernel(indices, autodecoder_embeddings):
    (B,) = indices.shape
    V, D = autodecoder_embeddings.shape
    info = plsc.get_sparse_core_info()
    NC, NS = info.num_cores, info.num_subcores
    NW = NC * NS
    b_per_w = B // NW
    n_idx_chunks = b_per_w // _CHUNK_IDX
    idx2d = indices.astype(jnp.int32).reshape(NW * n_idx_chunks, _CHUNK_IDX)
    half = V // 2
    lina = autodecoder_embeddings[:half].reshape(V // 4, 2 * D)
    linb = autodecoder_embeddings[half:].reshape(V // 4, 2 * D)
    return _gather_kernel(B, V, D, NW, b_per_w, n_idx_chunks)(idx2d, lina, linb)
